# baseline (speedup 1.0000x reference)
import functools

import jax
import jax.numpy as jnp
from jax.experimental import pallas as pl


def _fused_body(x_ref, noise_ref, wgn_ref, thr_ref, msk_ref,
                w1_ref, b1_ref, w2c_ref, out_ref, *, E, D):
    xf = x_ref[...]
    g8 = jnp.dot(xf, wgn_ref[...], preferred_element_type=jnp.float32)
    clean = g8[:, :E]
    rawn = g8[:, E:2 * E]
    noise_std = jax.nn.softplus(rawn) + 1e-2
    noisy = clean + noise_ref[...] * noise_std
    scores = noisy - thr_ref[...]
    sel = 0.5 * (jnp.sign(scores) + 1.0) * msk_ref[...]
    masked = jnp.where(sel > 0.0, clean, jnp.full_like(clean, -1e9))
    m = jnp.max(masked, axis=-1, keepdims=True)
    ex = jnp.exp(masked - m)
    sm = ex / jnp.sum(ex, axis=-1, keepdims=True)
    gates = sm * sel
    denom = jnp.clip(jnp.sum(gates, axis=-1, keepdims=True), 1e-9, None)
    gates = gates / denom
    xb = xf.astype(jnp.bfloat16)
    parts = []
    for e in range(E):
        he = jnp.dot(xb, w1_ref[e], preferred_element_type=jnp.float32)
        he = he + b1_ref[e][None, :]
        parts.append((jnp.maximum(he, 0.0) * gates[:, e:e + 1]).astype(jnp.bfloat16))
    parts.append(jnp.pad(gates, ((0, 0), (0, 128 - E))).astype(jnp.bfloat16))
    hg = jnp.concatenate(parts, axis=1)
    out_ref[...] = jnp.dot(hg, w2c_ref[...], preferred_element_type=jnp.float32)


def kernel(x, w_gate, w_noise, gate_threshold, experts_mask, noise, W1, b1, W2, b2):
    N, D = x.shape
    E = w_gate.shape[1]
    BT = 256
    gw = jnp.concatenate([w_gate, w_noise], axis=1)
    wgn = jnp.pad(gw, ((0, 0), (0, 128 - 2 * E)))
    w1b = W1.astype(jnp.bfloat16)
    w2c = jnp.concatenate(
        [W2.reshape(E * D, D), jnp.pad(b2, ((0, 128 - E), (0, 0)))],
        axis=0).astype(jnp.bfloat16)
    thr = gate_threshold.reshape(1, E)
    msk = experts_mask.reshape(1, E)

    grid = (N // BT,)
    body = functools.partial(_fused_body, E=E, D=D)
    return pl.pallas_call(
        body,
        grid=grid,
        in_specs=[
            pl.BlockSpec((BT, D), lambda i: (i, 0)),
            pl.BlockSpec((BT, E), lambda i: (i, 0)),
            pl.BlockSpec((D, 128), lambda i: (0, 0)),
            pl.BlockSpec((1, E), lambda i: (0, 0)),
            pl.BlockSpec((1, E), lambda i: (0, 0)),
            pl.BlockSpec((E, D, D), lambda i: (0, 0, 0)),
            pl.BlockSpec((E, D), lambda i: (0, 0)),
            pl.BlockSpec((E * D + 128, D), lambda i: (0, 0)),
        ],
        out_specs=pl.BlockSpec((BT, D), lambda i: (i, 0)),
        out_shape=jax.ShapeDtypeStruct((N, D), jnp.float32),
    )(x, noise, wgn, thr, msk, w1b, b1, w2c)


# speedup vs baseline: 1.0456x; 1.0456x over previous
import functools

import jax
import jax.numpy as jnp
from jax.experimental import pallas as pl


def _fused_body(x_ref, noise_ref, wgn_ref, thr_ref, msk_ref,
                w1_ref, b1_ref, w2c_ref, out_ref, *, E, D):
    xf = x_ref[...]
    g8 = jnp.dot(xf, wgn_ref[...], preferred_element_type=jnp.float32)
    clean = g8[:, :E]
    rawn = g8[:, E:2 * E]
    noise_std = jax.nn.softplus(rawn) + 1e-2
    noisy = clean + noise_ref[...] * noise_std
    scores = noisy - thr_ref[...]
    sel = 0.5 * (jnp.sign(scores) + 1.0) * msk_ref[...]
    masked = jnp.where(sel > 0.0, clean, jnp.full_like(clean, -1e9))
    m = jnp.max(masked, axis=-1, keepdims=True)
    ex = jnp.exp(masked - m)
    sm = ex / jnp.sum(ex, axis=-1, keepdims=True)
    gates = sm * sel
    denom = jnp.clip(jnp.sum(gates, axis=-1, keepdims=True), 1e-9, None)
    gates = gates / denom
    xb = xf.astype(jnp.bfloat16)
    parts = []
    for e in range(E):
        he = jnp.dot(xb, w1_ref[e], preferred_element_type=jnp.float32)
        he = he + b1_ref[e][None, :]
        parts.append((jnp.maximum(he, 0.0) * gates[:, e:e + 1]).astype(jnp.bfloat16))
    parts.append(jnp.pad(gates, ((0, 0), (0, 128 - E))).astype(jnp.bfloat16))
    hg = jnp.concatenate(parts, axis=1)
    out_ref[...] = jnp.dot(hg, w2c_ref[...], preferred_element_type=jnp.float32)


def kernel(x, w_gate, w_noise, gate_threshold, experts_mask, noise, W1, b1, W2, b2):
    N, D = x.shape
    E = w_gate.shape[1]
    BT = 1024
    gw = jnp.concatenate([w_gate, w_noise], axis=1)
    wgn = jnp.pad(gw, ((0, 0), (0, 128 - 2 * E)))
    w1b = W1.astype(jnp.bfloat16)
    w2c = jnp.concatenate(
        [W2.reshape(E * D, D), jnp.pad(b2, ((0, 128 - E), (0, 0)))],
        axis=0).astype(jnp.bfloat16)
    thr = gate_threshold.reshape(1, E)
    msk = experts_mask.reshape(1, E)

    grid = (N // BT,)
    body = functools.partial(_fused_body, E=E, D=D)
    return pl.pallas_call(
        body,
        grid=grid,
        in_specs=[
            pl.BlockSpec((BT, D), lambda i: (i, 0)),
            pl.BlockSpec((BT, E), lambda i: (i, 0)),
            pl.BlockSpec((D, 128), lambda i: (0, 0)),
            pl.BlockSpec((1, E), lambda i: (0, 0)),
            pl.BlockSpec((1, E), lambda i: (0, 0)),
            pl.BlockSpec((E, D, D), lambda i: (0, 0, 0)),
            pl.BlockSpec((E, D), lambda i: (0, 0)),
            pl.BlockSpec((E * D + 128, D), lambda i: (0, 0)),
        ],
        out_specs=pl.BlockSpec((BT, D), lambda i: (i, 0)),
        out_shape=jax.ShapeDtypeStruct((N, D), jnp.float32),
    )(x, noise, wgn, thr, msk, w1b, b1, w2c)


# speedup vs baseline: 1.0510x; 1.0052x over previous
import functools

import jax
import jax.numpy as jnp
from jax.experimental import pallas as pl


def _fused_body(x_ref, noise_ref, wgn_ref, thr_ref, msk_ref,
                w1_ref, b1_ref, w2c_ref, out_ref, *, E):
    xf = x_ref[...]
    g8 = jnp.dot(xf, wgn_ref[...], preferred_element_type=jnp.float32)
    clean = g8[:, :E]
    rawn = g8[:, E:2 * E]
    noise_std = jax.nn.softplus(rawn) + 1e-2
    noisy = clean + noise_ref[...] * noise_std
    scores = noisy - thr_ref[...]
    sel = 0.5 * (jnp.sign(scores) + 1.0) * msk_ref[...]
    masked = jnp.where(sel > 0.0, clean, jnp.full_like(clean, -1e9))
    m = jnp.max(masked, axis=-1, keepdims=True)
    ex = jnp.exp(masked - m)
    sm = ex / jnp.sum(ex, axis=-1, keepdims=True)
    gates = sm * sel
    denom = jnp.clip(jnp.sum(gates, axis=-1, keepdims=True), 1e-9, None)
    gates = gates / denom
    xb = xf.astype(jnp.bfloat16)
    parts = []
    for e in range(E):
        he = jnp.dot(xb, w1_ref[e], preferred_element_type=jnp.float32)
        he = he + b1_ref[e][None, :]
        parts.append((jnp.maximum(he, 0.0) * gates[:, e:e + 1]).astype(jnp.bfloat16))
    parts.append(jnp.pad(gates, ((0, 0), (0, 128 - E))).astype(jnp.bfloat16))
    hg = jnp.concatenate(parts, axis=1)
    out_ref[...] = jnp.dot(hg, w2c_ref[...], preferred_element_type=jnp.float32)


def kernel(x, w_gate, w_noise, gate_threshold, experts_mask, noise, W1, b1, W2, b2):
    N, D = x.shape
    E = w_gate.shape[1]
    BT = 1024
    gw = jnp.concatenate([w_gate, w_noise], axis=1)
    wgn = jnp.pad(gw, ((0, 0), (0, 128 - 2 * E)))
    w1b = W1.astype(jnp.bfloat16)
    w2c = jnp.concatenate(
        [W2.reshape(E * D, D), jnp.pad(b2, ((0, 128 - E), (0, 0)))],
        axis=0).astype(jnp.bfloat16)
    thr = gate_threshold.reshape(1, E)
    msk = experts_mask.reshape(1, E)

    grid = (N // BT,)
    body = functools.partial(_fused_body, E=E)
    return pl.pallas_call(
        body,
        grid=grid,
        in_specs=[
            pl.BlockSpec((BT, D), lambda i: (i, 0)),
            pl.BlockSpec((BT, E), lambda i: (i, 0)),
            pl.BlockSpec((D, 128), lambda i: (0, 0)),
            pl.BlockSpec((1, E), lambda i: (0, 0)),
            pl.BlockSpec((1, E), lambda i: (0, 0)),
            pl.BlockSpec((E, D, D), lambda i: (0, 0, 0)),
            pl.BlockSpec((E, D), lambda i: (0, 0)),
            pl.BlockSpec((E * D + 128, D), lambda i: (0, 0)),
        ],
        out_specs=pl.BlockSpec((BT, D), lambda i: (i, 0)),
        out_shape=jax.ShapeDtypeStruct((N, D), jnp.float32),
    )(x, noise, wgn, thr, msk, w1b, b1, w2c)
